# baseline (speedup 1.0000x reference)
import jax
import jax.numpy as jnp
from jax.experimental import pallas as pl

K = 32


def kernel(source, target):
    r0 = jnp.sum(target * target, axis=2, keepdims=True)
    r1 = jnp.sum(source * source, axis=2, keepdims=True)
    r1 = jnp.transpose(r1, (0, 2, 1))
    tb = target.astype(jnp.bfloat16).astype(jnp.float32)
    sb = source.astype(jnp.bfloat16).astype(jnp.float32)
    dot = jnp.matmul(tb, jnp.transpose(sb, (0, 2, 1)),
                     precision=jax.lax.Precision.HIGHEST)
    sq = r0 - 2.0 * dot + r1
    neg_vals, idx = jax.lax.top_k(-sq, K)
    b = source.shape[0]
    nt = target.shape[1]
    batch_idx = jnp.tile(jnp.arange(b).reshape(b, 1, 1), (1, nt, K))
    patches_idx = jnp.stack([batch_idx, idx], axis=-1)
    return patches_idx, -neg_vals


# speedup vs baseline: 8.7837x; 8.7837x over previous
import functools

import jax
import jax.numpy as jnp
from jax import lax
from jax.experimental import pallas as pl
from jax.experimental.pallas import tpu as pltpu
from jax.experimental.pallas import tpu_sc as plsc

NS = 8192
NT = 2048
K = 32
NW = 32
TPW = 128
CH = NS // 16
TG = 4

_F32MAX = 3.4028235e38


_DNUMS = lax.GatherDimensionNumbers(
    offset_dims=(), collapsed_slice_dims=(0,), start_index_map=(0,))


def _bcast(v, lane):
    idx = jnp.full((16,), lane, jnp.int32)
    return lax.gather(v, idx[:, None], _DNUMS, slice_sizes=(1,),
                      mode=lax.GatherScatterMode.PROMISE_IN_BOUNDS)


def _round_bf16(x):
    u = plsc.bitcast(x, jnp.int32)
    lsb = jnp.bitwise_and(lax.shift_right_logical(u, 16), 1)
    r = jnp.bitwise_and(u + 32767 + lsb, -65536)
    return plsc.bitcast(r, jnp.float32)


def _body(src_o, tgt_o, out_i, out_v,
          sxr, syr, szr, r1, dbuf, candv, candi,
          txr, tyr, tzr, tox, toy, toz, r0, outi_ref, outv_ref):
    wid = lax.axis_index("s") * 2 + lax.axis_index("c")
    b = wid // 16
    toff = (wid % 16) * TPW

    sbase = b * 3 * NS
    pltpu.sync_copy(src_o.at[pl.ds(sbase, NS)], dbuf.at[0])
    pltpu.sync_copy(src_o.at[pl.ds(sbase + NS, NS)], dbuf.at[1])
    pltpu.sync_copy(src_o.at[pl.ds(sbase + 2 * NS, NS)], dbuf.at[2])

    def r1_body(c, _):
        s = pl.ds(c * 16, 16)
        x = dbuf[0, s]
        y = dbuf[1, s]
        z = dbuf[2, s]
        r1[s] = (x * x + y * y) + z * z
        sxr[s] = _round_bf16(x)
        syr[s] = _round_bf16(y)
        szr[s] = _round_bf16(z)
        return 0
    lax.fori_loop(0, CH, r1_body, 0)

    tbase = b * 3 * NT + toff
    pltpu.sync_copy(tgt_o.at[pl.ds(tbase, TPW)], tox)
    pltpu.sync_copy(tgt_o.at[pl.ds(tbase + NT, TPW)], toy)
    pltpu.sync_copy(tgt_o.at[pl.ds(tbase + 2 * NT, TPW)], toz)

    def r0_body(c, _):
        s = pl.ds(c * 16, 16)
        x = tox[s]
        y = toy[s]
        z = toz[s]
        r0[s] = (x * x + y * y) + z * z
        txr[s] = _round_bf16(x)
        tyr[s] = _round_bf16(y)
        tzr[s] = _round_bf16(z)
        return 0
    lax.fori_loop(0, TPW // 16, r0_body, 0)

    iota16 = lax.iota(jnp.int32, 16)

    def tc_body(tc, _):
        ts = pl.ds(tc * 16, 16)
        tx16 = txr[ts]
        ty16 = tyr[ts]
        tz16 = tzr[ts]
        r016 = r0[ts]

        def gg_body(gg, _):
            lane0 = gg * TG
            txs = [_bcast(tx16, lane0 + j) for j in range(TG)]
            tys = [_bcast(ty16, lane0 + j) for j in range(TG)]
            tzs = [_bcast(tz16, lane0 + j) for j in range(TG)]
            r0s = [_bcast(r016, lane0 + j) for j in range(TG)]

            inf = jnp.full((16,), _F32MAX, jnp.float32)

            def a_body(c, carry):
                mins = list(carry)
                s = pl.ds(c * 16, 16)
                sx = sxr[s]
                sy = syr[s]
                sz = szr[s]
                r1c = r1[s]
                for j in range(TG):
                    dot = (sx * txs[j] + sy * tys[j]) + sz * tzs[j]
                    d = (r0s[j] - 2.0 * dot) + r1c
                    dbuf[j, s] = d
                    m1 = mins[2 * j]
                    m2 = mins[2 * j + 1]
                    mins[2 * j] = jnp.minimum(m1, d)
                    mins[2 * j + 1] = jnp.minimum(jnp.maximum(m1, d), m2)
                return tuple(mins)

            mins = lax.fori_loop(0, CH, a_body, (inf,) * (2 * TG))

            for j in range(TG):
                cm = plsc.cummax(mins[2 * j + 1])
                cutoff = _bcast(cm, 15)

                def b_body(c, offv):
                    s = pl.ds(c * 16, 16)
                    d = dbuf[j, s]
                    mask = d <= cutoff
                    mi = jnp.where(mask, 1, 0).astype(jnp.int32)
                    pos = jnp.cumsum(mi)
                    addr = offv + pos - mi
                    plsc.store_scatter(candv, [addr], d, mask=mask)
                    idxv = jnp.full((16,), c * 16, jnp.int32) + iota16
                    plsc.store_scatter(candi, [addr], idxv, mask=mask)
                    return offv + _bcast(pos, 15)

                offv = lax.fori_loop(0, CH, b_body,
                                     jnp.zeros((16,), jnp.int32))

                plsc.store_scatter(candv, [offv + iota16],
                                   jnp.full((16,), _F32MAX, jnp.float32))
                cnt = jnp.max(offv)
                trips = (cnt + 15) // 16

                def m_body(m, carry):
                    T0, T0i, T1, T1i = carry
                    s = pl.ds(m * 16, 16)
                    sv = candv[s]
                    si = candi[s]
                    sd, sdi = plsc.sort_key_val(sv, si, descending=True)
                    mk = T1 <= sd
                    cv = jnp.where(mk, T1, sd)
                    ci = jnp.where(mk, T1i, sdi)
                    cd, cdi = plsc.sort_key_val(cv, ci, descending=True)
                    mk2 = T0 <= cd
                    ev = jnp.where(mk2, T0, cd)
                    ei = jnp.where(mk2, T0i, cdi)
                    fv = jnp.where(mk2, cd, T0)
                    fi = jnp.where(mk2, cdi, T0i)
                    T0, T0i = plsc.sort_key_val(ev, ei)
                    T1, T1i = plsc.sort_key_val(fv, fi)
                    return (T0, T0i, T1, T1i)

                zi = jnp.zeros((16,), jnp.int32)
                T0, T0i, T1, T1i = lax.fori_loop(
                    0, trips, m_body, (inf, zi, inf, zi))

                tl = tc * 16 + lane0 + j
                out_base = tl * K
                oi = pl.ds(out_base, 16)
                oi2 = pl.ds(out_base + 16, 16)
                outi_ref[oi] = T0i
                outi_ref[oi2] = T1i
                outv_ref[oi] = T0
                outv_ref[oi2] = T1
            return 0

        lax.fori_loop(0, 16 // TG, gg_body, 0)
        return 0

    lax.fori_loop(0, TPW // 16, tc_body, 0)

    pltpu.sync_copy(outi_ref, out_i.at[pl.ds(wid * TPW * K, TPW * K)])
    pltpu.sync_copy(outv_ref, out_v.at[pl.ds(wid * TPW * K, TPW * K)])


def kernel(source, target):
    bsz, nt = target.shape[0], target.shape[1]
    src_o = jnp.transpose(source, (0, 2, 1)).reshape(-1)
    tgt_o = jnp.transpose(target, (0, 2, 1)).reshape(-1)

    mesh = plsc.VectorSubcoreMesh(core_axis_name="c", subcore_axis_name="s")
    run = pl.kernel(
        _body,
        out_type=(jax.ShapeDtypeStruct((bsz * nt * K,), jnp.int32),
                  jax.ShapeDtypeStruct((bsz * nt * K,), jnp.float32)),
        mesh=mesh,
        scratch_types=[
            pltpu.VMEM((NS,), jnp.float32),
            pltpu.VMEM((NS,), jnp.float32),
            pltpu.VMEM((NS,), jnp.float32),
            pltpu.VMEM((NS,), jnp.float32),
            pltpu.VMEM((TG, NS), jnp.float32),
            pltpu.VMEM((NS + 16,), jnp.float32),
            pltpu.VMEM((NS + 16,), jnp.int32),
            pltpu.VMEM((TPW,), jnp.float32),
            pltpu.VMEM((TPW,), jnp.float32),
            pltpu.VMEM((TPW,), jnp.float32),
            pltpu.VMEM((TPW,), jnp.float32),
            pltpu.VMEM((TPW,), jnp.float32),
            pltpu.VMEM((TPW,), jnp.float32),
            pltpu.VMEM((TPW,), jnp.float32),
            pltpu.VMEM((TPW * K,), jnp.int32),
            pltpu.VMEM((TPW * K,), jnp.float32),
        ],
        compiler_params=pltpu.CompilerParams(needs_layout_passes=False),
    )
    idx_flat, val_flat = run(src_o, tgt_o)
    idx = idx_flat.reshape(bsz, nt, K)
    vals = val_flat.reshape(bsz, nt, K)
    batch_idx = jnp.tile(
        jnp.arange(bsz, dtype=idx.dtype).reshape(bsz, 1, 1), (1, nt, K))
    patches_idx = jnp.stack([batch_idx, idx], axis=-1)
    return patches_idx, vals


# speedup vs baseline: 12.6681x; 1.4422x over previous
import functools

import jax
import jax.numpy as jnp
from jax import lax
from jax.experimental import pallas as pl
from jax.experimental.pallas import tpu as pltpu
from jax.experimental.pallas import tpu_sc as plsc

NS = 8192
NT = 2048
K = 32
NW = 32
TPW = 128
CH = NS // 16
TG = 4
LCAP = 257

_F32MAX = 3.4028235e38


_DNUMS = lax.GatherDimensionNumbers(
    offset_dims=(), collapsed_slice_dims=(0,), start_index_map=(0,))


def _bcast(v, lane):
    idx = jnp.full((16,), lane, jnp.int32)
    return lax.gather(v, idx[:, None], _DNUMS, slice_sizes=(1,),
                      mode=lax.GatherScatterMode.PROMISE_IN_BOUNDS)


def _round_bf16(x):
    u = plsc.bitcast(x, jnp.int32)
    lsb = jnp.bitwise_and(lax.shift_right_logical(u, 16), 1)
    r = jnp.bitwise_and(u + 32767 + lsb, -65536)
    return plsc.bitcast(r, jnp.float32)


def _body(src_o, tgt_o, out_i, out_v,
          sxr, syr, szr, r1, dbuf, candi,
          txr, tyr, tzr, tox, toy, toz, r0, outi_ref, outv_ref):
    wid = lax.axis_index("s") * 2 + lax.axis_index("c")
    b = wid // 16
    toff = (wid % 16) * TPW

    sbase = b * 3 * NS
    pltpu.sync_copy(src_o.at[pl.ds(sbase, NS)], dbuf.at[0])
    pltpu.sync_copy(src_o.at[pl.ds(sbase + NS, NS)], dbuf.at[1])
    pltpu.sync_copy(src_o.at[pl.ds(sbase + 2 * NS, NS)], dbuf.at[2])

    def r1_body(c, _):
        s = pl.ds(c * 16, 16)
        x = dbuf[0, s]
        y = dbuf[1, s]
        z = dbuf[2, s]
        r1[s] = (x * x + y * y) + z * z
        sxr[s] = _round_bf16(x)
        syr[s] = _round_bf16(y)
        szr[s] = _round_bf16(z)
        return 0
    lax.fori_loop(0, CH, r1_body, 0)

    tbase = b * 3 * NT + toff
    pltpu.sync_copy(tgt_o.at[pl.ds(tbase, TPW)], tox)
    pltpu.sync_copy(tgt_o.at[pl.ds(tbase + NT, TPW)], toy)
    pltpu.sync_copy(tgt_o.at[pl.ds(tbase + 2 * NT, TPW)], toz)

    def r0_body(c, _):
        s = pl.ds(c * 16, 16)
        x = tox[s]
        y = toy[s]
        z = toz[s]
        r0[s] = (x * x + y * y) + z * z
        txr[s] = _round_bf16(x) * 2.0
        tyr[s] = _round_bf16(y) * 2.0
        tzr[s] = _round_bf16(z) * 2.0
        return 0
    lax.fori_loop(0, TPW // 16, r0_body, 0)

    iota16 = lax.iota(jnp.int32, 16)
    lanebase = iota16 * LCAP

    def tc_body(tc, _):
        ts = pl.ds(tc * 16, 16)
        tx16 = txr[ts]
        ty16 = tyr[ts]
        tz16 = tzr[ts]
        r016 = r0[ts]

        def gg_body(gg, _):
            lane0 = gg * TG
            txs = [_bcast(tx16, lane0 + j) for j in range(TG)]
            tys = [_bcast(ty16, lane0 + j) for j in range(TG)]
            tzs = [_bcast(tz16, lane0 + j) for j in range(TG)]
            r0s = [_bcast(r016, lane0 + j) for j in range(TG)]

            inf = jnp.full((16,), _F32MAX, jnp.float32)

            def a_body(c, carry):
                mins = list(carry)
                s = pl.ds(c * 16, 16)
                sx = sxr[s]
                sy = syr[s]
                sz = szr[s]
                r1c = r1[s]
                for j in range(TG):
                    dot2 = (sx * txs[j] + sy * tys[j]) + sz * tzs[j]
                    d = (r0s[j] - dot2) + r1c
                    dbuf[j, s] = d
                    m1 = mins[2 * j]
                    m2 = mins[2 * j + 1]
                    mins[2 * j] = jnp.minimum(m1, d)
                    mins[2 * j + 1] = jnp.minimum(jnp.maximum(m1, d), m2)
                return tuple(mins)

            mins = lax.fori_loop(0, CH, a_body, (inf,) * (2 * TG))

            cutoffs = []
            for j in range(TG):
                cm = plsc.cummax(mins[2 * j + 1])
                cutoffs.append(_bcast(cm, 15))

            def b_body(c, carry):
                lanecnts = list(carry)
                s = pl.ds(c * 16, 16)
                idxv = jnp.full((16,), c * 16, jnp.int32) + iota16
                for j in range(TG):
                    d = dbuf[j, s]
                    mask = d <= cutoffs[j]
                    mi = jnp.where(mask, 1, 0).astype(jnp.int32)
                    lc = lanecnts[j]
                    addr = (lanebase + jnp.minimum(lc, LCAP - 1)
                            + j * (16 * LCAP))
                    plsc.store_scatter(candi, [addr], idxv, mask=mask)
                    lanecnts[j] = lc + mi
                return tuple(lanecnts)

            lanecnts = lax.fori_loop(
                0, CH, b_body, (jnp.zeros((16,), jnp.int32),) * TG)

            for j in range(TG):
                lanecnt = jnp.minimum(lanecnts[j], LCAP)
                trips = jnp.max(lanecnt)
                jsplat = jnp.full((16,), j, jnp.int32)

                def m_body(m, carry):
                    T0, T0i, T1, T1i, rv = carry
                    addr = lanebase + rv + j * (16 * LCAP)
                    valid = rv < lanecnt
                    si = plsc.load_gather(candi, [addr], mask=valid)
                    sv = plsc.load_gather(dbuf, [jsplat, si], mask=valid)
                    sv = jnp.where(valid, sv, _F32MAX)
                    sd, sdi = plsc.sort_key_val(sv, si, descending=True)
                    mk = T1 <= sd
                    cv = jnp.where(mk, T1, sd)
                    ci = jnp.where(mk, T1i, sdi)
                    cd, cdi = plsc.sort_key_val(cv, ci, descending=True)
                    mk2 = T0 <= cd
                    ev = jnp.where(mk2, T0, cd)
                    ei = jnp.where(mk2, T0i, cdi)
                    fv = jnp.where(mk2, cd, T0)
                    fi = jnp.where(mk2, cdi, T0i)
                    T0, T0i = plsc.sort_key_val(ev, ei)
                    T1, T1i = plsc.sort_key_val(fv, fi)
                    return (T0, T0i, T1, T1i, rv + 1)

                zi = jnp.zeros((16,), jnp.int32)
                T0, T0i, T1, T1i, _ = lax.fori_loop(
                    0, trips, m_body, (inf, zi, inf, zi, zi))

                tl = tc * 16 + lane0 + j
                out_base = tl * K
                oi = pl.ds(out_base, 16)
                oi2 = pl.ds(out_base + 16, 16)
                outi_ref[oi] = T0i
                outi_ref[oi2] = T1i
                outv_ref[oi] = T0
                outv_ref[oi2] = T1
            return 0

        lax.fori_loop(0, 16 // TG, gg_body, 0)
        return 0

    lax.fori_loop(0, TPW // 16, tc_body, 0)

    pltpu.sync_copy(outi_ref, out_i.at[pl.ds(wid * TPW * K, TPW * K)])
    pltpu.sync_copy(outv_ref, out_v.at[pl.ds(wid * TPW * K, TPW * K)])


def kernel(source, target):
    bsz, nt = target.shape[0], target.shape[1]
    src_o = jnp.transpose(source, (0, 2, 1)).reshape(-1)
    tgt_o = jnp.transpose(target, (0, 2, 1)).reshape(-1)

    mesh = plsc.VectorSubcoreMesh(core_axis_name="c", subcore_axis_name="s")
    run = pl.kernel(
        _body,
        out_type=(jax.ShapeDtypeStruct((bsz * nt * K,), jnp.int32),
                  jax.ShapeDtypeStruct((bsz * nt * K,), jnp.float32)),
        mesh=mesh,
        scratch_types=[
            pltpu.VMEM((NS,), jnp.float32),
            pltpu.VMEM((NS,), jnp.float32),
            pltpu.VMEM((NS,), jnp.float32),
            pltpu.VMEM((NS,), jnp.float32),
            pltpu.VMEM((TG, NS), jnp.float32),
            pltpu.VMEM((TG * 16 * LCAP,), jnp.int32),
            pltpu.VMEM((TPW,), jnp.float32),
            pltpu.VMEM((TPW,), jnp.float32),
            pltpu.VMEM((TPW,), jnp.float32),
            pltpu.VMEM((TPW,), jnp.float32),
            pltpu.VMEM((TPW,), jnp.float32),
            pltpu.VMEM((TPW,), jnp.float32),
            pltpu.VMEM((TPW,), jnp.float32),
            pltpu.VMEM((TPW * K,), jnp.int32),
            pltpu.VMEM((TPW * K,), jnp.float32),
        ],
        compiler_params=pltpu.CompilerParams(needs_layout_passes=False),
    )
    idx_flat, val_flat = run(src_o, tgt_o)
    idx = idx_flat.reshape(bsz, nt, K)
    vals = val_flat.reshape(bsz, nt, K)
    batch_idx = jnp.tile(
        jnp.arange(bsz, dtype=idx.dtype).reshape(bsz, 1, 1), (1, nt, K))
    patches_idx = jnp.stack([batch_idx, idx], axis=-1)
    return patches_idx, vals


# speedup vs baseline: 22.3664x; 1.7656x over previous
import functools

import jax
import jax.numpy as jnp
from jax import lax
from jax.experimental import pallas as pl
from jax.experimental.pallas import tpu as pltpu
from jax.experimental.pallas import tpu_sc as plsc

NS = 8192
NT = 2048
K = 32
NW = 32
TPW = 128
CH = NS // 16
TG = 4
LCAP = 257

_F32MAX = 3.4028235e38


_DNUMS = lax.GatherDimensionNumbers(
    offset_dims=(), collapsed_slice_dims=(0,), start_index_map=(0,))


def _bcast(v, lane):
    idx = jnp.full((16,), lane, jnp.int32)
    return lax.gather(v, idx[:, None], _DNUMS, slice_sizes=(1,),
                      mode=lax.GatherScatterMode.PROMISE_IN_BOUNDS)


def _round_bf16(x):
    u = plsc.bitcast(x, jnp.int32)
    lsb = jnp.bitwise_and(lax.shift_right_logical(u, 16), 1)
    r = jnp.bitwise_and(u + 32767 + lsb, -65536)
    return plsc.bitcast(r, jnp.float32)


def _body(src_o, tgt_o, out_i, out_v,
          sxr, syr, szr, r1, dbuf, candi,
          txr, tyr, tzr, tox, toy, toz, r0, outi_ref, outv_ref):
    wid = lax.axis_index("s") * 2 + lax.axis_index("c")
    b = wid // 16
    toff = (wid % 16) * TPW

    sbase = b * 3 * NS
    pltpu.sync_copy(src_o.at[pl.ds(sbase, NS)], dbuf.at[0])
    pltpu.sync_copy(src_o.at[pl.ds(sbase + NS, NS)], dbuf.at[1])
    pltpu.sync_copy(src_o.at[pl.ds(sbase + 2 * NS, NS)], dbuf.at[2])

    def r1_body(c, _):
        s = pl.ds(c * 16, 16)
        x = dbuf[0, s]
        y = dbuf[1, s]
        z = dbuf[2, s]
        r1[s] = (x * x + y * y) + z * z
        sxr[s] = _round_bf16(x)
        syr[s] = _round_bf16(y)
        szr[s] = _round_bf16(z)
        return 0
    lax.fori_loop(0, CH, r1_body, 0)

    tbase = b * 3 * NT + toff
    pltpu.sync_copy(tgt_o.at[pl.ds(tbase, TPW)], tox)
    pltpu.sync_copy(tgt_o.at[pl.ds(tbase + NT, TPW)], toy)
    pltpu.sync_copy(tgt_o.at[pl.ds(tbase + 2 * NT, TPW)], toz)

    def r0_body(c, _):
        s = pl.ds(c * 16, 16)
        x = tox[s]
        y = toy[s]
        z = toz[s]
        r0[s] = (x * x + y * y) + z * z
        txr[s] = _round_bf16(x) * 2.0
        tyr[s] = _round_bf16(y) * 2.0
        tzr[s] = _round_bf16(z) * 2.0
        return 0
    lax.fori_loop(0, TPW // 16, r0_body, 0)

    iota16 = lax.iota(jnp.int32, 16)
    lanebase = iota16 * LCAP

    def tc_body(tc, _):
        ts = pl.ds(tc * 16, 16)
        tx16 = txr[ts]
        ty16 = tyr[ts]
        tz16 = tzr[ts]
        r016 = r0[ts]

        def gg_body(gg, _):
            lane0 = gg * TG
            txs = [_bcast(tx16, lane0 + j) for j in range(TG)]
            tys = [_bcast(ty16, lane0 + j) for j in range(TG)]
            tzs = [_bcast(tz16, lane0 + j) for j in range(TG)]
            r0s = [_bcast(r016, lane0 + j) for j in range(TG)]

            inf = jnp.full((16,), _F32MAX, jnp.float32)

            def a_body(c, carry):
                mins = list(carry)
                s = pl.ds(c * 16, 16)
                sx = sxr[s]
                sy = syr[s]
                sz = szr[s]
                r1c = r1[s]
                for j in range(TG):
                    dot2 = (sx * txs[j] + sy * tys[j]) + sz * tzs[j]
                    d = (r0s[j] - dot2) + r1c
                    dbuf[j, s] = d
                    m1 = mins[2 * j]
                    m2 = mins[2 * j + 1]
                    mins[2 * j] = jnp.minimum(m1, d)
                    mins[2 * j + 1] = jnp.minimum(jnp.maximum(m1, d), m2)
                return tuple(mins)

            mins = lax.fori_loop(0, CH, a_body, (inf,) * (2 * TG))

            cutoffs = []
            for j in range(TG):
                cm = plsc.cummax(mins[2 * j + 1])
                cutoffs.append(_bcast(cm, 15))

            def b_body(c, carry):
                lanecnts = list(carry)
                s = pl.ds(c * 16, 16)
                idxv = jnp.full((16,), c * 16, jnp.int32) + iota16
                dsl = [dbuf[j, s] for j in range(TG)]
                masks = [dsl[j] <= cutoffs[j] for j in range(TG)]
                addrs = [lanebase + j * (16 * LCAP)
                         + jnp.bitwise_and(lanecnts[j], 255)
                         for j in range(TG)]
                news = [lanecnts[j]
                        + jnp.where(masks[j], 1, 0).astype(jnp.int32)
                        for j in range(TG)]
                for j in range(TG):
                    plsc.store_scatter(candi, [addrs[j]], idxv,
                                       mask=masks[j])
                return tuple(news)

            lanecnts = lax.fori_loop(
                0, CH, b_body, (jnp.zeros((16,), jnp.int32),) * TG)

            for j in range(TG):
                lanecnt = jnp.minimum(lanecnts[j], 256)
                trips = jnp.max(lanecnt)
                jsplat = jnp.full((16,), j, jnp.int32)

                def m_body(m, carry):
                    T0, T0i, T1, T1i, rv = carry
                    addr = lanebase + rv + j * (16 * LCAP)
                    valid = rv < lanecnt
                    si = plsc.load_gather(candi, [addr], mask=valid)
                    sv = plsc.load_gather(dbuf, [jsplat, si], mask=valid)
                    sv = jnp.where(valid, sv, _F32MAX)
                    sd, sdi = plsc.sort_key_val(sv, si, descending=True)
                    mk = T1 <= sd
                    cv = jnp.where(mk, T1, sd)
                    ci = jnp.where(mk, T1i, sdi)
                    cd, cdi = plsc.sort_key_val(cv, ci, descending=True)
                    mk2 = T0 <= cd
                    ev = jnp.where(mk2, T0, cd)
                    ei = jnp.where(mk2, T0i, cdi)
                    fv = jnp.where(mk2, cd, T0)
                    fi = jnp.where(mk2, cdi, T0i)
                    T0, T0i = plsc.sort_key_val(ev, ei)
                    T1, T1i = plsc.sort_key_val(fv, fi)
                    return (T0, T0i, T1, T1i, rv + 1)

                zi = jnp.zeros((16,), jnp.int32)
                T0, T0i, T1, T1i, _ = lax.fori_loop(
                    0, trips, m_body, (inf, zi, inf, zi, zi))

                tl = tc * 16 + lane0 + j
                out_base = tl * K
                oi = pl.ds(out_base, 16)
                oi2 = pl.ds(out_base + 16, 16)
                outi_ref[oi] = T0i
                outi_ref[oi2] = T1i
                outv_ref[oi] = T0
                outv_ref[oi2] = T1
            return 0

        lax.fori_loop(0, 16 // TG, gg_body, 0)
        return 0

    lax.fori_loop(0, TPW // 16, tc_body, 0)

    pltpu.sync_copy(outi_ref, out_i.at[pl.ds(wid * TPW * K, TPW * K)])
    pltpu.sync_copy(outv_ref, out_v.at[pl.ds(wid * TPW * K, TPW * K)])


def kernel(source, target):
    bsz, nt = target.shape[0], target.shape[1]
    src_o = jnp.transpose(source, (0, 2, 1)).reshape(-1)
    tgt_o = jnp.transpose(target, (0, 2, 1)).reshape(-1)

    mesh = plsc.VectorSubcoreMesh(core_axis_name="c", subcore_axis_name="s")
    run = pl.kernel(
        _body,
        out_type=(jax.ShapeDtypeStruct((bsz * nt * K,), jnp.int32),
                  jax.ShapeDtypeStruct((bsz * nt * K,), jnp.float32)),
        mesh=mesh,
        scratch_types=[
            pltpu.VMEM((NS,), jnp.float32),
            pltpu.VMEM((NS,), jnp.float32),
            pltpu.VMEM((NS,), jnp.float32),
            pltpu.VMEM((NS,), jnp.float32),
            pltpu.VMEM((TG, NS), jnp.float32),
            pltpu.VMEM((TG * 16 * LCAP,), jnp.int32),
            pltpu.VMEM((TPW,), jnp.float32),
            pltpu.VMEM((TPW,), jnp.float32),
            pltpu.VMEM((TPW,), jnp.float32),
            pltpu.VMEM((TPW,), jnp.float32),
            pltpu.VMEM((TPW,), jnp.float32),
            pltpu.VMEM((TPW,), jnp.float32),
            pltpu.VMEM((TPW,), jnp.float32),
            pltpu.VMEM((TPW * K,), jnp.int32),
            pltpu.VMEM((TPW * K,), jnp.float32),
        ],
        compiler_params=pltpu.CompilerParams(needs_layout_passes=False),
    )
    idx_flat, val_flat = run(src_o, tgt_o)
    idx = idx_flat.reshape(bsz, nt, K)
    vals = val_flat.reshape(bsz, nt, K)
    batch_idx = jnp.tile(
        jnp.arange(bsz, dtype=idx.dtype).reshape(bsz, 1, 1), (1, nt, K))
    patches_idx = jnp.stack([batch_idx, idx], axis=-1)
    return patches_idx, vals


# speedup vs baseline: 32.6238x; 1.4586x over previous
import functools

import jax
import jax.numpy as jnp
from jax import lax
from jax.experimental import pallas as pl
from jax.experimental.pallas import tpu as pltpu
from jax.experimental.pallas import tpu_sc as plsc

NS = 8192
NT = 2048
K = 32
NW = 32
TPW = 128
CH = NS // 16
TG = 4
LCAP = 257

_F32MAX = 3.4028235e38


_DNUMS = lax.GatherDimensionNumbers(
    offset_dims=(), collapsed_slice_dims=(0,), start_index_map=(0,))


def _bcast(v, lane):
    idx = jnp.full((16,), lane, jnp.int32)
    return lax.gather(v, idx[:, None], _DNUMS, slice_sizes=(1,),
                      mode=lax.GatherScatterMode.PROMISE_IN_BOUNDS)


def _round_bf16(x):
    u = plsc.bitcast(x, jnp.int32)
    lsb = jnp.bitwise_and(lax.shift_right_logical(u, 16), 1)
    r = jnp.bitwise_and(u + 32767 + lsb, -65536)
    return plsc.bitcast(r, jnp.float32)


def _body(src_o, tgt_o, out_i, out_v,
          sxr, syr, szr, r1, dbuf, candi,
          txr, tyr, tzr, tox, toy, toz, r0, outi_ref, outv_ref):
    wid = lax.axis_index("s") * 2 + lax.axis_index("c")
    b = wid // 16
    toff = (wid % 16) * TPW

    sbase = b * 3 * NS
    pltpu.sync_copy(src_o.at[pl.ds(sbase, NS)], dbuf.at[0])
    pltpu.sync_copy(src_o.at[pl.ds(sbase + NS, NS)], dbuf.at[1])
    pltpu.sync_copy(src_o.at[pl.ds(sbase + 2 * NS, NS)], dbuf.at[2])

    def r1_body(c, _):
        s = pl.ds(c * 16, 16)
        x = dbuf[0, s]
        y = dbuf[1, s]
        z = dbuf[2, s]
        r1[s] = (x * x + y * y) + z * z
        sxr[s] = _round_bf16(x)
        syr[s] = _round_bf16(y)
        szr[s] = _round_bf16(z)
        return 0
    lax.fori_loop(0, CH, r1_body, 0)

    tbase = b * 3 * NT + toff
    pltpu.sync_copy(tgt_o.at[pl.ds(tbase, TPW)], tox)
    pltpu.sync_copy(tgt_o.at[pl.ds(tbase + NT, TPW)], toy)
    pltpu.sync_copy(tgt_o.at[pl.ds(tbase + 2 * NT, TPW)], toz)

    def r0_body(c, _):
        s = pl.ds(c * 16, 16)
        x = tox[s]
        y = toy[s]
        z = toz[s]
        r0[s] = (x * x + y * y) + z * z
        txr[s] = _round_bf16(x) * 2.0
        tyr[s] = _round_bf16(y) * 2.0
        tzr[s] = _round_bf16(z) * 2.0
        return 0
    lax.fori_loop(0, TPW // 16, r0_body, 0)

    iota16 = lax.iota(jnp.int32, 16)
    lanebase = iota16 * LCAP

    def tc_body(tc, _):
        ts = pl.ds(tc * 16, 16)
        tx16 = txr[ts]
        ty16 = tyr[ts]
        tz16 = tzr[ts]
        r016 = r0[ts]

        def gg_body(gg, _):
            lane0 = gg * TG
            txs = [_bcast(tx16, lane0 + j) for j in range(TG)]
            tys = [_bcast(ty16, lane0 + j) for j in range(TG)]
            tzs = [_bcast(tz16, lane0 + j) for j in range(TG)]
            r0s = [_bcast(r016, lane0 + j) for j in range(TG)]

            inf = jnp.full((16,), _F32MAX, jnp.float32)

            def a_body(c, carry=None):
                mins = list(carry)
                s = pl.ds(c * 16, 16)
                sx = sxr[s]
                sy = syr[s]
                sz = szr[s]
                r1c = r1[s]
                for j in range(TG):
                    dot2 = (sx * txs[j] + sy * tys[j]) + sz * tzs[j]
                    d = (r0s[j] - dot2) + r1c
                    dbuf[j, s] = d
                    m1 = mins[2 * j]
                    m2 = mins[2 * j + 1]
                    mins[2 * j] = jnp.minimum(m1, d)
                    mins[2 * j + 1] = jnp.minimum(jnp.maximum(m1, d), m2)
                return tuple(mins)

            mins = plsc.parallel_loop(0, CH, carry=(inf,) * (2 * TG))(a_body)

            cutoffs = []
            for j in range(TG):
                cm = plsc.cummax(mins[2 * j + 1])
                cutoffs.append(_bcast(cm, 15))

            def b_body(c, carry):
                lanecnts = list(carry)
                s = pl.ds(c * 16, 16)
                idxv = jnp.full((16,), c * 16, jnp.int32) + iota16
                dsl = [dbuf[j, s] for j in range(TG)]
                masks = [dsl[j] <= cutoffs[j] for j in range(TG)]
                addrs = [lanebase + j * (16 * LCAP)
                         + jnp.bitwise_and(lanecnts[j], 255)
                         for j in range(TG)]
                news = [lanecnts[j]
                        + jnp.where(masks[j], 1, 0).astype(jnp.int32)
                        for j in range(TG)]
                for j in range(TG):
                    plsc.store_scatter(candi, [addrs[j]], idxv,
                                       mask=masks[j])
                return tuple(news)

            lanecnts = plsc.parallel_loop(
                0, CH, carry=(jnp.zeros((16,), jnp.int32),) * TG)(b_body)

            for j in range(TG):
                lanecnt = jnp.minimum(lanecnts[j], 256)
                trips = jnp.max(lanecnt)
                jsplat = jnp.full((16,), j, jnp.int32)

                def m_body(m, carry):
                    T0, T0i, T1, T1i, rv = carry
                    addr = lanebase + rv + j * (16 * LCAP)
                    valid = rv < lanecnt
                    si = plsc.load_gather(candi, [addr], mask=valid)
                    sv = plsc.load_gather(dbuf, [jsplat, si], mask=valid)
                    sv = jnp.where(valid, sv, _F32MAX)
                    sd, sdi = plsc.sort_key_val(sv, si, descending=True)
                    mk = T1 <= sd
                    cv = jnp.where(mk, T1, sd)
                    ci = jnp.where(mk, T1i, sdi)
                    cd, cdi = plsc.sort_key_val(cv, ci, descending=True)
                    mk2 = T0 <= cd
                    ev = jnp.where(mk2, T0, cd)
                    ei = jnp.where(mk2, T0i, cdi)
                    fv = jnp.where(mk2, cd, T0)
                    fi = jnp.where(mk2, cdi, T0i)
                    T0, T0i = plsc.sort_key_val(ev, ei)
                    T1, T1i = plsc.sort_key_val(fv, fi)
                    return (T0, T0i, T1, T1i, rv + 1)

                zi = jnp.zeros((16,), jnp.int32)
                T0, T0i, T1, T1i, _ = lax.fori_loop(
                    0, trips, m_body, (inf, zi, inf, zi, zi))

                tl = tc * 16 + lane0 + j
                out_base = tl * K
                oi = pl.ds(out_base, 16)
                oi2 = pl.ds(out_base + 16, 16)
                outi_ref[oi] = T0i
                outi_ref[oi2] = T1i
                outv_ref[oi] = T0
                outv_ref[oi2] = T1
            return 0

        lax.fori_loop(0, 16 // TG, gg_body, 0)
        return 0

    lax.fori_loop(0, TPW // 16, tc_body, 0)

    pltpu.sync_copy(outi_ref, out_i.at[pl.ds(wid * TPW * K, TPW * K)])
    pltpu.sync_copy(outv_ref, out_v.at[pl.ds(wid * TPW * K, TPW * K)])


def kernel(source, target):
    bsz, nt = target.shape[0], target.shape[1]
    src_o = jnp.transpose(source, (0, 2, 1)).reshape(-1)
    tgt_o = jnp.transpose(target, (0, 2, 1)).reshape(-1)

    mesh = plsc.VectorSubcoreMesh(core_axis_name="c", subcore_axis_name="s")
    run = pl.kernel(
        _body,
        out_type=(jax.ShapeDtypeStruct((bsz * nt * K,), jnp.int32),
                  jax.ShapeDtypeStruct((bsz * nt * K,), jnp.float32)),
        mesh=mesh,
        scratch_types=[
            pltpu.VMEM((NS,), jnp.float32),
            pltpu.VMEM((NS,), jnp.float32),
            pltpu.VMEM((NS,), jnp.float32),
            pltpu.VMEM((NS,), jnp.float32),
            pltpu.VMEM((TG, NS), jnp.float32),
            pltpu.VMEM((TG * 16 * LCAP,), jnp.int32),
            pltpu.VMEM((TPW,), jnp.float32),
            pltpu.VMEM((TPW,), jnp.float32),
            pltpu.VMEM((TPW,), jnp.float32),
            pltpu.VMEM((TPW,), jnp.float32),
            pltpu.VMEM((TPW,), jnp.float32),
            pltpu.VMEM((TPW,), jnp.float32),
            pltpu.VMEM((TPW,), jnp.float32),
            pltpu.VMEM((TPW * K,), jnp.int32),
            pltpu.VMEM((TPW * K,), jnp.float32),
        ],
        compiler_params=pltpu.CompilerParams(needs_layout_passes=False),
    )
    idx_flat, val_flat = run(src_o, tgt_o)
    idx = idx_flat.reshape(bsz, nt, K)
    vals = val_flat.reshape(bsz, nt, K)
    batch_idx = jnp.tile(
        jnp.arange(bsz, dtype=idx.dtype).reshape(bsz, 1, 1), (1, nt, K))
    patches_idx = jnp.stack([batch_idx, idx], axis=-1)
    return patches_idx, vals
